# baseline (speedup 1.0000x reference)
import jax
import jax.numpy as jnp
from jax.experimental import pallas as pl
from jax.experimental.pallas import tpu as pltpu


def _add_body(pos_ref, x_ref, pe_ref, o_ref):
    o_ref[...] = x_ref[...] + pe_ref[...]


def kernel(x, batch_positions, pe):
    B, T, d_model, H, W = x.shape
    BT = B * T
    HW = H * W
    x3 = x.reshape(BT, d_model, HW)
    pos = batch_positions.reshape(BT)
    pe3 = pe.reshape(pe.shape[0], d_model, 1)

    grid_spec = pltpu.PrefetchScalarGridSpec(
        num_scalar_prefetch=1,
        grid=(BT,),
        in_specs=[
            pl.BlockSpec((1, d_model, HW), lambda i, pos_ref: (i, 0, 0)),
            pl.BlockSpec((1, d_model, 1), lambda i, pos_ref: (pos_ref[i], 0, 0)),
        ],
        out_specs=pl.BlockSpec((1, d_model, HW), lambda i, pos_ref: (i, 0, 0)),
    )
    out = pl.pallas_call(
        _add_body,
        grid_spec=grid_spec,
        out_shape=jax.ShapeDtypeStruct((BT, d_model, HW), jnp.float32),
    )(pos, x3, pe3)
    return out.reshape(B, T, d_model, H, W)


# speedup vs baseline: 1.2677x; 1.2677x over previous
import functools

import jax
import jax.numpy as jnp
from jax import lax
from jax.experimental import pallas as pl
from jax.experimental.pallas import tpu as pltpu
from jax.experimental.pallas import tpu_sc as plsc

_ROWS_PER_WORKER = 8


@functools.lru_cache(maxsize=None)
def _make_sc_gather(num_rows, d_model, max_len):
    info = plsc.get_sparse_core_info()
    num_cores = info.num_cores
    mesh = plsc.VectorSubcoreMesh(core_axis_name="c", subcore_axis_name="s")
    active = num_rows // _ROWS_PER_WORKER

    @functools.partial(
        pl.kernel,
        mesh=mesh,
        out_type=jax.ShapeDtypeStruct((num_rows, d_model), jnp.float32),
        scratch_types=[
            pltpu.VMEM((_ROWS_PER_WORKER,), jnp.int32),
            pltpu.VMEM((_ROWS_PER_WORKER, d_model), jnp.float32),
            pltpu.SemaphoreType.DMA,
        ],
    )
    def gather(pe_hbm, idx_hbm, out_hbm, idx_v, rows_v, sem):
        wid = lax.axis_index("s") * num_cores + lax.axis_index("c")

        @pl.when(wid < active)
        def _():
            base = wid * _ROWS_PER_WORKER
            pltpu.sync_copy(idx_hbm.at[pl.ds(base, _ROWS_PER_WORKER)], idx_v)
            pltpu.async_copy(pe_hbm.at[idx_v], rows_v, sem).wait()
            pltpu.sync_copy(rows_v, out_hbm.at[pl.ds(base, _ROWS_PER_WORKER)])

    return gather


def _add_body(x_ref, t_ref, o_ref):
    o_ref[...] = x_ref[...] + t_ref[...]


def kernel(x, batch_positions, pe):
    B, T, d_model, H, W = x.shape
    BT = B * T
    HW = H * W
    x3 = x.reshape(BT, d_model, HW)
    pos = batch_positions.reshape(BT)

    table = _make_sc_gather(BT, d_model, pe.shape[0])(pe, pos)
    table3 = table.reshape(BT, d_model, 1)

    G = 8
    out = pl.pallas_call(
        _add_body,
        grid=(BT // G,),
        in_specs=[
            pl.BlockSpec((G, d_model, HW), lambda i: (i, 0, 0)),
            pl.BlockSpec((G, d_model, 1), lambda i: (i, 0, 0)),
        ],
        out_specs=pl.BlockSpec((G, d_model, HW), lambda i: (i, 0, 0)),
        out_shape=jax.ShapeDtypeStruct((BT, d_model, HW), jnp.float32),
    )(x3, table3)
    return out.reshape(B, T, d_model, H, W)


# speedup vs baseline: 1.3229x; 1.0435x over previous
import functools

import jax
import jax.numpy as jnp
from jax import lax
from jax.experimental import pallas as pl
from jax.experimental.pallas import tpu as pltpu
from jax.experimental.pallas import tpu_sc as plsc

_ROWS_PER_WORKER = 8


@functools.lru_cache(maxsize=None)
def _make_sc_gather(num_rows, d_model, max_len):
    info = plsc.get_sparse_core_info()
    num_cores = info.num_cores
    mesh = plsc.VectorSubcoreMesh(core_axis_name="c", subcore_axis_name="s")
    active = num_rows // _ROWS_PER_WORKER

    @functools.partial(
        pl.kernel,
        mesh=mesh,
        out_type=jax.ShapeDtypeStruct((num_rows, d_model), jnp.float32),
        scratch_types=[
            pltpu.VMEM((_ROWS_PER_WORKER,), jnp.int32),
            pltpu.VMEM((_ROWS_PER_WORKER, d_model), jnp.float32),
            pltpu.SemaphoreType.DMA,
        ],
    )
    def gather(pe_hbm, idx_hbm, out_hbm, idx_v, rows_v, sem):
        wid = lax.axis_index("s") * num_cores + lax.axis_index("c")

        @pl.when(wid < active)
        def _():
            base = wid * _ROWS_PER_WORKER
            pltpu.sync_copy(idx_hbm.at[pl.ds(base, _ROWS_PER_WORKER)], idx_v)
            pltpu.async_copy(pe_hbm.at[idx_v], rows_v, sem).wait()
            pltpu.sync_copy(rows_v, out_hbm.at[pl.ds(base, _ROWS_PER_WORKER)])

    return gather


def _add_body(x_ref, t_ref, o_ref):
    G = x_ref.shape[0]
    vt = t_ref[...].T
    for g in range(G):
        o_ref[g] = x_ref[g] + vt[:, g : g + 1]


def kernel(x, batch_positions, pe):
    B, T, d_model, H, W = x.shape
    BT = B * T
    HW = H * W
    x3 = x.reshape(BT, d_model, HW)
    pos = batch_positions.reshape(BT)

    table = _make_sc_gather(BT, d_model, pe.shape[0])(pe, pos)

    G = 8
    out = pl.pallas_call(
        _add_body,
        grid=(BT // G,),
        in_specs=[
            pl.BlockSpec((G, d_model, HW), lambda i: (i, 0, 0)),
            pl.BlockSpec((G, d_model), lambda i: (i, 0)),
        ],
        out_specs=pl.BlockSpec((G, d_model, HW), lambda i: (i, 0, 0)),
        out_shape=jax.ShapeDtypeStruct((BT, d_model, HW), jnp.float32),
    )(x3, table)
    return out.reshape(B, T, d_model, H, W)


# speedup vs baseline: 1.3241x; 1.0009x over previous
import functools

import jax
import jax.numpy as jnp
from jax import lax
from jax.experimental import pallas as pl
from jax.experimental.pallas import tpu as pltpu
from jax.experimental.pallas import tpu_sc as plsc

_ROWS_PER_WORKER = 8
_CHUNK = 2
_NBUF = 8


@functools.lru_cache(maxsize=None)
def _make_sc_gather(num_rows, d_model, max_len):
    info = plsc.get_sparse_core_info()
    num_cores = info.num_cores
    mesh = plsc.VectorSubcoreMesh(core_axis_name="c", subcore_axis_name="s")
    active = num_rows // _ROWS_PER_WORKER

    @functools.partial(
        pl.kernel,
        mesh=mesh,
        out_type=jax.ShapeDtypeStruct((num_rows, d_model), jnp.float32),
        scratch_types=[
            pltpu.VMEM((_ROWS_PER_WORKER,), jnp.int32),
            pltpu.VMEM((_ROWS_PER_WORKER, d_model), jnp.float32),
            pltpu.SemaphoreType.DMA,
        ],
    )
    def gather(pe_hbm, idx_hbm, out_hbm, idx_v, rows_v, sem):
        wid = lax.axis_index("s") * num_cores + lax.axis_index("c")

        @pl.when(wid < active)
        def _():
            base = wid * _ROWS_PER_WORKER
            pltpu.sync_copy(idx_hbm.at[pl.ds(base, _ROWS_PER_WORKER)], idx_v)
            pltpu.async_copy(pe_hbm.at[idx_v], rows_v, sem).wait()
            pltpu.sync_copy(rows_v, out_hbm.at[pl.ds(base, _ROWS_PER_WORKER)])

    return gather


def _add_body(t_ref, x_hbm, o_hbm, in_bufs, out_bufs, in_sems, out_sems):
    n_chunks = x_hbm.shape[0] // _CHUNK

    def in_copy(i, b):
        return pltpu.make_async_copy(
            x_hbm.at[pl.ds(i * _CHUNK, _CHUNK)], in_bufs.at[b], in_sems.at[b]
        )

    def out_copy(i, b):
        return pltpu.make_async_copy(
            out_bufs.at[b], o_hbm.at[pl.ds(i * _CHUNK, _CHUNK)], out_sems.at[b]
        )

    for j in range(_NBUF):
        in_copy(j, j).start()

    def step(i, _):
        b = lax.rem(i, _NBUF)
        in_copy(i, b).wait()

        @pl.when(i >= _NBUF)
        def _():
            out_copy(i - _NBUF, b).wait()

        vt = t_ref[pl.ds(i * _CHUNK, _CHUNK), :].T
        for g in range(_CHUNK):
            out_bufs[b, g] = in_bufs[b, g] + vt[:, g : g + 1]

        out_copy(i, b).start()

        @pl.when(i + _NBUF < n_chunks)
        def _():
            in_copy(i + _NBUF, b).start()

        return _

    lax.fori_loop(0, n_chunks, step, None)

    def drain(i, _):
        out_copy(i, lax.rem(i, _NBUF)).wait()
        return _

    lax.fori_loop(n_chunks - _NBUF, n_chunks, drain, None)


def kernel(x, batch_positions, pe):
    B, T, d_model, H, W = x.shape
    BT = B * T
    HW = H * W
    x3 = x.reshape(BT, d_model, HW)
    pos = batch_positions.reshape(BT)

    table = _make_sc_gather(BT, d_model, pe.shape[0])(pe, pos)

    out = pl.pallas_call(
        _add_body,
        in_specs=[
            pl.BlockSpec(memory_space=pltpu.MemorySpace.VMEM),
            pl.BlockSpec(memory_space=pltpu.MemorySpace.HBM),
        ],
        out_specs=pl.BlockSpec(memory_space=pltpu.MemorySpace.HBM),
        out_shape=jax.ShapeDtypeStruct((BT, d_model, HW), jnp.float32),
        scratch_shapes=[
            pltpu.VMEM((_NBUF, _CHUNK, d_model, HW), jnp.float32),
            pltpu.VMEM((_NBUF, _CHUNK, d_model, HW), jnp.float32),
            pltpu.SemaphoreType.DMA((_NBUF,)),
            pltpu.SemaphoreType.DMA((_NBUF,)),
        ],
    )(table, x3)
    return out.reshape(B, T, d_model, H, W)
